# baseline (device time: 54773 ns/iter reference)
import jax
import jax.numpy as jnp
from jax import lax
from jax.experimental import pallas as pl
from jax.experimental.pallas import tpu as pltpu

B = 16
H = 16
D = 64
BS = 16
NP_LOC = 128
NB = 128
NKEY = NP_LOC * BS
BH = B * H
HD = H * D
SCALE = D ** -0.5


def kernel(Q, K, V, bt, lens):
    lens_row = lens.reshape(1, B)
    bt_t = jnp.transpose(bt)
    qq = jnp.transpose(Q[:, 0, :, :], (2, 0, 1)).reshape(D, BH)
    km = K.reshape(NKEY, HD).astype(jnp.bfloat16)
    vm = V.reshape(NKEY, HD).astype(jnp.bfloat16)

    def body(qq_ref, km_ref, vm_ref, btt_ref, lens_ref, out_ref,
             comm_ref, send_sem, recv_sem):
        my_x = lax.axis_index("x")
        my_y = lax.axis_index("y")
        my_z = lax.axis_index("z")
        peer = (1 - my_x, my_y, my_z)

        btt3 = btt_ref[:, :][:, None, :]
        pids = lax.broadcasted_iota(jnp.int32, (NB, NP_LOC, B), 1) \
            + my_x * NP_LOC
        jidx = lax.broadcasted_iota(jnp.int32, (NB, 1, B), 0)
        valid = jidx < lens_ref[:, :].reshape(1, 1, B)
        hit = jnp.logical_and(btt3 == pids, valid)
        cnt_t = jnp.sum(jnp.where(hit, 1.0, 0.0), axis=0)
        ek_t = jnp.where(
            lax.broadcasted_iota(jnp.int32, (NKEY, NP_LOC), 0) // BS
            == lax.broadcasted_iota(jnp.int32, (NKEY, NP_LOC), 1),
            1.0, 0.0).astype(jnp.bfloat16)
        cb = lax.dot_general(
            ek_t, cnt_t.astype(jnp.bfloat16),
            (((1,), (0,)), ((), ())),
            preferred_element_type=jnp.float32)
        en_t = jnp.where(
            lax.broadcasted_iota(jnp.int32, (B, BH), 0)
            == lax.broadcasted_iota(jnp.int32, (B, BH), 1) // H,
            1.0, 0.0).astype(jnp.bfloat16)
        ck_t = lax.dot_general(
            cb.astype(jnp.bfloat16), en_t,
            (((1,), (0,)), ((), ())),
            preferred_element_type=jnp.float32)

        qtile = jnp.broadcast_to(
            qq_ref[:, :].astype(jnp.bfloat16)[None, :, :], (H, D, BH))
        wmask = (lax.broadcasted_iota(jnp.int32, (H, D, BH), 0)
                 == lax.broadcasted_iota(jnp.int32, (H, D, BH), 2) % H)
        w = jnp.where(wmask, qtile, 0.0).reshape(HD, BH)

        s_t = lax.dot_general(
            km_ref[:, :], w, (((1,), (0,)), ((), ())),
            preferred_element_type=jnp.float32,
        ) * SCALE
        m = jnp.max(s_t, axis=0, keepdims=True)
        p = ck_t * jnp.exp(s_t - m)
        l = jnp.sum(p, axis=0, keepdims=True)
        p_t = jnp.transpose(p.astype(jnp.bfloat16))
        o_big = lax.dot_general(
            p_t, vm_ref[:, :], (((1,), (0,)), ((), ())),
            preferred_element_type=jnp.float32,
        )
        osel = (lax.broadcasted_iota(jnp.int32, (BH, HD), 1) // D
                == lax.broadcasted_iota(jnp.int32, (BH, HD), 0) % H)
        o_m = jnp.where(osel, o_big, 0.0)
        acc = o_m[:, 0:D]
        for hh in range(1, H):
            acc = acc + o_m[:, hh * D:(hh + 1) * D]

        comm_ref[0, :, 0:D] = acc
        comm_ref[0, :, D:D + 1] = jnp.transpose(m)
        comm_ref[0, :, D + 1:D + 2] = jnp.transpose(l)

        barrier_sem = pltpu.get_barrier_semaphore()
        pl.semaphore_signal(barrier_sem, inc=1, device_id=peer,
                            device_id_type=pl.DeviceIdType.MESH)
        pl.semaphore_wait(barrier_sem, 1)

        rdma = pltpu.make_async_remote_copy(
            src_ref=comm_ref.at[0],
            dst_ref=comm_ref.at[1],
            send_sem=send_sem,
            recv_sem=recv_sem,
            device_id=peer,
            device_id_type=pl.DeviceIdType.MESH,
        )
        rdma.start()
        rdma.wait()

        acc1 = comm_ref[0, :, 0:D]
        m1 = comm_ref[0, :, D:D + 1]
        l1 = comm_ref[0, :, D + 1:D + 2]
        acc2 = comm_ref[1, :, 0:D]
        m2 = comm_ref[1, :, D:D + 1]
        l2 = comm_ref[1, :, D + 1:D + 2]
        m_new = jnp.maximum(m1, m2)
        a1 = jnp.exp(m1 - m_new)
        a2 = jnp.exp(m2 - m_new)
        l_tot = l1 * a1 + l2 * a2
        out = (acc1 * a1 + acc2 * a2) / l_tot
        out_ref[:, 0, :, :] = out.reshape(B, H, D)

    return pl.pallas_call(
        body,
        out_shape=jax.ShapeDtypeStruct((B, 1, H, D), jnp.float32),
        in_specs=[
            pl.BlockSpec(memory_space=pltpu.VMEM),
            pl.BlockSpec(memory_space=pltpu.VMEM),
            pl.BlockSpec(memory_space=pltpu.VMEM),
            pl.BlockSpec(memory_space=pltpu.VMEM),
            pl.BlockSpec(memory_space=pltpu.VMEM),
        ],
        out_specs=pl.BlockSpec(memory_space=pltpu.VMEM),
        scratch_shapes=[
            pltpu.VMEM((2, BH, 128), jnp.float32),
            pltpu.SemaphoreType.DMA,
            pltpu.SemaphoreType.DMA,
        ],
        compiler_params=pltpu.CompilerParams(collective_id=0),
    )(qq, km, vm, bt_t, lens_row)


# device time: 27563 ns/iter; 1.9872x vs baseline; 1.9872x over previous
import jax
import jax.numpy as jnp
from jax import lax
from jax.experimental import pallas as pl
from jax.experimental.pallas import tpu as pltpu

B = 16
H = 16
D = 64
BS = 16
NP_LOC = 128
NB = 128
SCALE = D ** -0.5


def kernel(Q, K, V, bt, lens):
    kt = jnp.transpose(K, (1, 2, 3, 0))
    vt = jnp.transpose(V, (1, 2, 3, 0))
    q_t = jnp.transpose(Q[:, 0, :, :], (1, 0, 2))
    lens2d = lens.reshape(B, 1)

    def body(q_ref, kt_ref, vt_ref, bt_ref, lens_ref, out_ref,
             comm_ref, send_sem, recv_sem):
        my_x = lax.axis_index("x")
        my_y = lax.axis_index("y")
        my_z = lax.axis_index("z")
        peer = (1 - my_x, my_y, my_z)

        bt3 = bt_ref[:, :][:, :, None]
        pids = lax.broadcasted_iota(jnp.int32, (B, NB, NP_LOC), 2) \
            + my_x * NP_LOC
        jidx = lax.broadcasted_iota(jnp.int32, (B, NB, 1), 1)
        valid = jidx < lens_ref[:, :][:, None, :]
        hit = jnp.logical_and(bt3 == pids, valid)
        cnt = jnp.sum(jnp.where(hit, 1.0, 0.0), axis=1)

        SH = BS * H
        q3 = jnp.broadcast_to(
            (q_ref[:, :, :] * SCALE).astype(jnp.bfloat16)[None, :, :, :],
            (BS, H, B, D)).reshape(SH, B, D)
        k3 = kt_ref[:, :, :, :].astype(jnp.bfloat16).reshape(SH, D, NP_LOC)
        s3 = lax.dot_general(
            q3, k3, (((2,), (1,)), ((0,), (0,))),
            preferred_element_type=jnp.float32)
        m_hb = jnp.max(jnp.max(s3, axis=2).reshape(BS, H, B), axis=0)
        m_bc = jnp.broadcast_to(m_hb[None, :, :], (BS, H, B)) \
            .reshape(SH, B)[:, :, None]
        p3 = cnt[None, :, :] * jnp.exp(s3 - m_bc)
        l_hb = jnp.sum(jnp.sum(p3, axis=2).reshape(BS, H, B), axis=0)
        pt3 = jnp.transpose(p3.astype(jnp.bfloat16), (0, 2, 1))
        v3 = vt_ref[:, :, :, :].astype(jnp.bfloat16).reshape(SH, D, NP_LOC)
        o3 = lax.dot_general(
            v3, pt3, (((2,), (1,)), ((0,), (0,))),
            preferred_element_type=jnp.float32)
        acc = jnp.sum(o3.reshape(BS, H, D, B), axis=0)

        comm_ref[0, :, 0:D, :] = acc
        comm_ref[0, :, D:D + 1, :] = m_hb[:, None, :]
        comm_ref[0, :, D + 1:D + 2, :] = l_hb[:, None, :]

        barrier_sem = pltpu.get_barrier_semaphore()
        pl.semaphore_signal(barrier_sem, inc=1, device_id=peer,
                            device_id_type=pl.DeviceIdType.MESH)
        pl.semaphore_wait(barrier_sem, 1)

        rdma = pltpu.make_async_remote_copy(
            src_ref=comm_ref.at[0],
            dst_ref=comm_ref.at[1],
            send_sem=send_sem,
            recv_sem=recv_sem,
            device_id=peer,
            device_id_type=pl.DeviceIdType.MESH,
        )
        rdma.start()
        rdma.wait()

        acc1 = comm_ref[0, :, 0:D, :]
        m1 = comm_ref[0, :, D:D + 1, :]
        l1 = comm_ref[0, :, D + 1:D + 2, :]
        acc2 = comm_ref[1, :, 0:D, :]
        m2 = comm_ref[1, :, D:D + 1, :]
        l2 = comm_ref[1, :, D + 1:D + 2, :]
        m_new = jnp.maximum(m1, m2)
        a1 = jnp.exp(m1 - m_new)
        a2 = jnp.exp(m2 - m_new)
        l_tot = l1 * a1 + l2 * a2
        res = (acc1 * a1 + acc2 * a2) / l_tot
        out_ref[:, 0, :, :] = jnp.transpose(res, (2, 0, 1))

    return pl.pallas_call(
        body,
        out_shape=jax.ShapeDtypeStruct((B, 1, H, D), jnp.float32),
        in_specs=[
            pl.BlockSpec(memory_space=pltpu.VMEM),
            pl.BlockSpec(memory_space=pltpu.VMEM),
            pl.BlockSpec(memory_space=pltpu.VMEM),
            pl.BlockSpec(memory_space=pltpu.VMEM),
            pl.BlockSpec(memory_space=pltpu.VMEM),
        ],
        out_specs=pl.BlockSpec(memory_space=pltpu.VMEM),
        scratch_shapes=[
            pltpu.VMEM((2, H, D + 8, B), jnp.float32),
            pltpu.SemaphoreType.DMA,
            pltpu.SemaphoreType.DMA,
        ],
        compiler_params=pltpu.CompilerParams(collective_id=0),
    )(q_t, kt, vt, bt, lens2d)
